# baseline (speedup 1.0000x reference)
import functools

import jax
import jax.numpy as jnp
from jax.experimental import pallas as pl
from jax.experimental.pallas import tpu as pltpu

K_NEIGHBORS = 10
N_POINTS = 4096


def _leaky(x):
    return jnp.where(x >= 0, x, 0.2 * x)


def _dot(a, b, dims):
    return jax.lax.dot_general(a, b, (dims, ((), ())),
                               preferred_element_type=jnp.float32)


def _edge_stage_kernel(x_ref, wa_ref, ba_ref, ga_ref, bea_ref,
                       wb_ref, bb_ref, gb_ref, beb_ref, o_ref, *, m, k):
    n_pts = x_ref.shape[1]
    x_all = x_ref[0]
    base = pl.program_id(1) * m
    xt = x_ref[0, pl.ds(base, m), :]

    xx_all = jnp.sum(x_all * x_all, axis=1)
    xx_t = jnp.sum(xt * xt, axis=1)
    dist = (2.0 * _dot(xt, x_all, ((1,), (1,)))
            - xx_t[:, None] - xx_all[None, :])

    iota = jax.lax.broadcasted_iota(jnp.int32, (m, n_pts), 1)
    big_i = jnp.int32(n_pts)
    big_f = jnp.float32(1e30)
    d = dist
    rows = []
    neg = jnp.float32(-jnp.inf)
    for i in range(k):
        am = jnp.argmax(d, axis=1)
        sel = iota == am[:, None]
        rows.append(_dot(jnp.where(sel, 1.0, 0.0), x_all, ((1,), (0,))))
        if i + 1 < k:
            d = jnp.where(sel, neg, d)

    feat = jnp.concatenate(rows, axis=0)
    xc = jnp.concatenate([xt] * k, axis=0)
    e = jnp.concatenate([feat - xc, xc], axis=1)

    y = _dot(e, wa_ref[...], ((1,), (1,))) + ba_ref[...]
    y = _leaky(y * ga_ref[...] + bea_ref[...])
    z = _dot(y, wb_ref[...], ((1,), (1,))) + bb_ref[...]
    z = _leaky(z * gb_ref[...] + beb_ref[...])
    o2 = z.shape[1]
    o_ref[0] = jnp.max(z.reshape(k, m, o2), axis=0)


def _edge_stage(xt, wa, ba, ga, bea, wb, bb, gb, beb, *, m):
    b, n, c = xt.shape
    o1, _ = wa.shape
    o2, _ = wb.shape
    row = lambda v: v.reshape(1, -1)
    spec_full = lambda s: pl.BlockSpec(s, lambda bi, ni: (0,) * len(s))
    return pl.pallas_call(
        functools.partial(_edge_stage_kernel, m=m, k=K_NEIGHBORS),
        grid=(b, n // m),
        in_specs=[
            pl.BlockSpec((1, n, c), lambda bi, ni: (bi, 0, 0)),
            spec_full((o1, 2 * c)), spec_full((1, o1)), spec_full((1, o1)),
            spec_full((1, o1)),
            spec_full((o2, o1)), spec_full((1, o2)), spec_full((1, o2)),
            spec_full((1, o2)),
        ],
        out_specs=pl.BlockSpec((1, m, o2), lambda bi, ni: (bi, ni, 0)),
        out_shape=jax.ShapeDtypeStruct((b, n, o2), jnp.float32),
        compiler_params=pltpu.CompilerParams(
            dimension_semantics=("parallel", "parallel")),
    )(xt, wa, row(ba), row(ga), row(bea), wb, row(bb), row(gb), row(beb))


def _stage3_kernel(x1_ref, x2_ref, wa1_ref, wa2_ref, ba_ref, ga_ref, bea_ref,
                   wb_ref, bb_ref, gb_ref, beb_ref, o_ref):
    x1 = x1_ref[0]
    x2 = x2_ref[0]
    y = (_dot(x1, wa1_ref[...], ((1,), (1,)))
         + _dot(x2, wa2_ref[...], ((1,), (1,))) + ba_ref[...])
    y = _leaky(y * ga_ref[...] + bea_ref[...])
    z = _dot(y, wb_ref[...], ((1,), (1,))) + bb_ref[...]
    z = _leaky(z * gb_ref[...] + beb_ref[...])
    o_ref[0] = jnp.max(z, axis=0, keepdims=True)


def _stage3(x1, x2, wa, ba, ga, bea, wb, bb, gb, beb):
    b, n, c1 = x1.shape
    c2 = x2.shape[2]
    o1 = wa.shape[0]
    o2 = wb.shape[0]
    row = lambda v: v.reshape(1, -1)
    spec_full = lambda s: pl.BlockSpec(s, lambda bi: (0,) * len(s))
    return pl.pallas_call(
        _stage3_kernel,
        grid=(b,),
        in_specs=[
            pl.BlockSpec((1, n, c1), lambda bi: (bi, 0, 0)),
            pl.BlockSpec((1, n, c2), lambda bi: (bi, 0, 0)),
            spec_full((o1, c1)), spec_full((o1, c2)), spec_full((1, o1)),
            spec_full((1, o1)), spec_full((1, o1)),
            spec_full((o2, o1)), spec_full((1, o2)), spec_full((1, o2)),
            spec_full((1, o2)),
        ],
        out_specs=pl.BlockSpec((1, 1, o2), lambda bi: (bi, 0, 0)),
        out_shape=jax.ShapeDtypeStruct((b, 1, o2), jnp.float32),
        compiler_params=pltpu.CompilerParams(
            dimension_semantics=("parallel",)),
    )(x1, x2, wa[:, :c1], wa[:, c1:], row(ba), row(ga), row(bea),
      wb, row(bb), row(gb), row(beb)).reshape(b, o2)


def _head_kernel(g_ref, w1_ref, b1_ref, g1_ref, be1_ref, w2_ref, b2_ref,
                 o_ref):
    y = _dot(g_ref[...], w1_ref[...], ((1,), (1,))) + b1_ref[...]
    y = _leaky(y * g1_ref[...] + be1_ref[...])
    z = _dot(y, w2_ref[...], ((1,), (1,))) + b2_ref[...]
    s = z - jnp.max(z, axis=1, keepdims=True)
    o_ref[...] = s - jnp.log(jnp.sum(jnp.exp(s), axis=1, keepdims=True))


def _head(gfeat, l1w, l1b, gl1, bel1, l2w, l2b):
    b, c = gfeat.shape
    h1 = l1w.shape[0]
    h2 = l2w.shape[0]
    row = lambda v: v.reshape(1, -1)
    spec = lambda s: pl.BlockSpec(s, lambda: (0,) * len(s))
    return pl.pallas_call(
        _head_kernel,
        grid=(),
        in_specs=[spec((b, c)), spec((h1, c)), spec((1, h1)), spec((1, h1)),
                  spec((1, h1)), spec((h2, h1)), spec((1, h2))],
        out_specs=spec((b, h2)),
        out_shape=jax.ShapeDtypeStruct((b, h2), jnp.float32),
    )(gfeat, l1w, row(l1b), row(gl1), row(bel1), l2w, row(l2b))


def kernel(x, w1a, b1a, g1a, be1a, w1b, b1b, g1b, be1b,
           w2a, b2a, g2a, be2a, w2b, b2b, g2b, be2b,
           w3a, b3a, g3a, be3a, w3b, b3b, g3b, be3b,
           l1w, l1b, gl1, bel1, l2w, l2b):
    xt = jnp.transpose(x, (0, 2, 1))
    x1 = _edge_stage(xt, w1a, b1a, g1a, be1a, w1b, b1b, g1b, be1b, m=1024)
    x2 = _edge_stage(x1, w2a, b2a, g2a, be2a, w2b, b2b, g2b, be2b, m=512)
    gfeat = _stage3(x1, x2, w3a, b3a, g3a, be3a, w3b, b3b, g3b, be3b)
    return _head(gfeat, l1w, l1b, gl1, bel1, l2w, l2b)


# speedup vs baseline: 1.0884x; 1.0884x over previous
import functools

import jax
import jax.numpy as jnp
from jax.experimental import pallas as pl
from jax.experimental.pallas import tpu as pltpu

K_NEIGHBORS = 10
N_POINTS = 4096


def _leaky(x):
    return jnp.where(x >= 0, x, 0.2 * x)


def _dot(a, b, dims):
    return jax.lax.dot_general(a, b, (dims, ((), ())),
                               preferred_element_type=jnp.float32)


def _edge_stage_kernel(x_ref, wa_ref, ba_ref, ga_ref, bea_ref,
                       wb_ref, bb_ref, gb_ref, beb_ref, o_ref, *, m, k):
    n_pts = x_ref.shape[1]
    x_all = x_ref[0]
    base = pl.program_id(1) * m
    xt = x_ref[0, pl.ds(base, m), :]

    xx_all = jnp.sum(x_all * x_all, axis=1)
    xx_t = jnp.sum(xt * xt, axis=1)
    dist = (2.0 * _dot(xt, x_all, ((1,), (1,)))
            - xx_t[:, None] - xx_all[None, :])

    iota = jax.lax.broadcasted_iota(jnp.int32, (m, n_pts), 1)
    big_i = jnp.int32(n_pts)
    big_f = jnp.float32(1e30)
    d = dist
    rows = []
    neg = jnp.float32(-jnp.inf)
    for i in range(k):
        am = jnp.argmax(d, axis=1)
        sel = iota == am[:, None]
        rows.append(_dot(jnp.where(sel, 1.0, 0.0), x_all, ((1,), (0,))))
        if i + 1 < k:
            d = jnp.where(sel, neg, d)

    feat = jnp.concatenate(rows, axis=0)
    xc = jnp.concatenate([xt] * k, axis=0)
    e = jnp.concatenate([feat - xc, xc], axis=1)

    y = _dot(e, wa_ref[...], ((1,), (1,))) + ba_ref[...]
    y = _leaky(y * ga_ref[...] + bea_ref[...])
    z = _dot(y, wb_ref[...], ((1,), (1,))) + bb_ref[...]
    z = _leaky(z * gb_ref[...] + beb_ref[...])
    o2 = z.shape[1]
    o_ref[0] = jnp.max(z.reshape(k, m, o2), axis=0)


def _edge_stage(xt, wa, ba, ga, bea, wb, bb, gb, beb, *, m):
    b, n, c = xt.shape
    o1, _ = wa.shape
    o2, _ = wb.shape
    row = lambda v: v.reshape(1, -1)
    spec_full = lambda s: pl.BlockSpec(s, lambda bi, ni: (0,) * len(s))
    return pl.pallas_call(
        functools.partial(_edge_stage_kernel, m=m, k=K_NEIGHBORS),
        grid=(b, n // m),
        in_specs=[
            pl.BlockSpec((1, n, c), lambda bi, ni: (bi, 0, 0)),
            spec_full((o1, 2 * c)), spec_full((1, o1)), spec_full((1, o1)),
            spec_full((1, o1)),
            spec_full((o2, o1)), spec_full((1, o2)), spec_full((1, o2)),
            spec_full((1, o2)),
        ],
        out_specs=pl.BlockSpec((1, m, o2), lambda bi, ni: (bi, ni, 0)),
        out_shape=jax.ShapeDtypeStruct((b, n, o2), jnp.float32),
        compiler_params=pltpu.CompilerParams(
            dimension_semantics=("parallel", "parallel")),
    )(xt, wa, row(ba), row(ga), row(bea), wb, row(bb), row(gb), row(beb))


def _stage3_kernel(x1_ref, x2_ref, wa1_ref, wa2_ref, ba_ref, ga_ref, bea_ref,
                   wb_ref, bb_ref, gb_ref, beb_ref, o_ref):
    x1 = x1_ref[0]
    x2 = x2_ref[0]
    y = (_dot(x1, wa1_ref[...], ((1,), (1,)))
         + _dot(x2, wa2_ref[...], ((1,), (1,))) + ba_ref[...])
    y = _leaky(y * ga_ref[...] + bea_ref[...])
    z = _dot(y, wb_ref[...], ((1,), (1,))) + bb_ref[...]
    z = _leaky(z * gb_ref[...] + beb_ref[...])
    o_ref[0] = jnp.max(z, axis=0, keepdims=True)


def _stage3(x1, x2, wa, ba, ga, bea, wb, bb, gb, beb):
    b, n, c1 = x1.shape
    c2 = x2.shape[2]
    o1 = wa.shape[0]
    o2 = wb.shape[0]
    row = lambda v: v.reshape(1, -1)
    spec_full = lambda s: pl.BlockSpec(s, lambda bi: (0,) * len(s))
    return pl.pallas_call(
        _stage3_kernel,
        grid=(b,),
        in_specs=[
            pl.BlockSpec((1, n, c1), lambda bi: (bi, 0, 0)),
            pl.BlockSpec((1, n, c2), lambda bi: (bi, 0, 0)),
            spec_full((o1, c1)), spec_full((o1, c2)), spec_full((1, o1)),
            spec_full((1, o1)), spec_full((1, o1)),
            spec_full((o2, o1)), spec_full((1, o2)), spec_full((1, o2)),
            spec_full((1, o2)),
        ],
        out_specs=pl.BlockSpec((1, 1, o2), lambda bi: (bi, 0, 0)),
        out_shape=jax.ShapeDtypeStruct((b, 1, o2), jnp.float32),
        compiler_params=pltpu.CompilerParams(
            dimension_semantics=("parallel",)),
    )(x1, x2, wa[:, :c1], wa[:, c1:], row(ba), row(ga), row(bea),
      wb, row(bb), row(gb), row(beb)).reshape(b, o2)


def _head_kernel(g_ref, w1_ref, b1_ref, g1_ref, be1_ref, w2_ref, b2_ref,
                 o_ref):
    y = _dot(g_ref[...], w1_ref[...], ((1,), (1,))) + b1_ref[...]
    y = _leaky(y * g1_ref[...] + be1_ref[...])
    z = _dot(y, w2_ref[...], ((1,), (1,))) + b2_ref[...]
    s = z - jnp.max(z, axis=1, keepdims=True)
    o_ref[...] = s - jnp.log(jnp.sum(jnp.exp(s), axis=1, keepdims=True))


def _head(gfeat, l1w, l1b, gl1, bel1, l2w, l2b):
    b, c = gfeat.shape
    h1 = l1w.shape[0]
    h2 = l2w.shape[0]
    row = lambda v: v.reshape(1, -1)
    spec = lambda s: pl.BlockSpec(s, lambda: (0,) * len(s))
    return pl.pallas_call(
        _head_kernel,
        grid=(),
        in_specs=[spec((b, c)), spec((h1, c)), spec((1, h1)), spec((1, h1)),
                  spec((1, h1)), spec((h2, h1)), spec((1, h2))],
        out_specs=spec((b, h2)),
        out_shape=jax.ShapeDtypeStruct((b, h2), jnp.float32),
    )(gfeat, l1w, row(l1b), row(gl1), row(bel1), l2w, row(l2b))


def kernel(x, w1a, b1a, g1a, be1a, w1b, b1b, g1b, be1b,
           w2a, b2a, g2a, be2a, w2b, b2b, g2b, be2b,
           w3a, b3a, g3a, be3a, w3b, b3b, g3b, be3b,
           l1w, l1b, gl1, bel1, l2w, l2b):
    xt = jnp.transpose(x, (0, 2, 1))
    x1 = _edge_stage(xt, w1a, b1a, g1a, be1a, w1b, b1b, g1b, be1b, m=256)
    x2 = _edge_stage(x1, w2a, b2a, g2a, be2a, w2b, b2b, g2b, be2b, m=256)
    gfeat = _stage3(x1, x2, w3a, b3a, g3a, be3a, w3b, b3b, g3b, be3b)
    return _head(gfeat, l1w, l1b, gl1, bel1, l2w, l2b)
